# baseline (speedup 1.0000x reference)
import functools

import jax
import jax.numpy as jnp
from jax import lax
from jax.experimental import pallas as pl
from jax.experimental.pallas import tpu as pltpu
from jax.experimental.pallas import tpu_sc as plsc

_SEQ = 200
_D = 64
_L = 16
_V = 16
_NC = 2
_NS = 16
_NW = _NC * _NS
_RPT = 128


def _cmp_exchange(ka, va, kb, vb, asc):
    m = (ka <= kb) if asc else (ka >= kb)
    kl = jnp.where(m, ka, kb)
    vl = jnp.where(m, va, vb)
    kh = jnp.where(m, kb, ka)
    vh = jnp.where(m, vb, va)
    return kl, vl, kh, vh


def _bitonic_sort(keys, vals):
    for v in range(_V):
        keys[v], vals[v] = plsc.sort_key_val(keys[v], vals[v],
                                             descending=bool(v & 1))
    for vk in (2, 4, 8, 16):
        vj = vk // 2
        while vj >= 1:
            for v in range(_V):
                if (v & vj) == 0:
                    p = v | vj
                    asc = (v & vk) == 0
                    keys[v], vals[v], keys[p], vals[p] = _cmp_exchange(
                        keys[v], vals[v], keys[p], vals[p], asc)
            vj //= 2
        for v in range(_V):
            asc = (v & vk) == 0
            keys[v], vals[v] = plsc.sort_key_val(keys[v], vals[v],
                                                 descending=not asc)
    return keys, vals


def _row_order(ts_buf, row_base, iota, idxm1, flag_v):
    inf = jnp.float32(jnp.inf)
    keys = []
    vals = []
    for g in range(_V):
        if g < 12:
            k = ts_buf[pl.ds(row_base + g * _L, _L)]
        elif g == 12:
            k = ts_buf[pl.ds(row_base + 12 * _L, _L)]
            k = jnp.where(iota < 8, k, inf)
        else:
            k = jnp.full((_L,), inf, jnp.float32)
        keys.append(k)
        vals.append(iota + g * _L)

    keys, vals = _bitonic_sort(keys, vals)

    comp = []
    carry_seg = jnp.int32(0)
    prev_last = jnp.float32(-jnp.inf)
    for g in range(_V):
        shifted = keys[g].at[idxm1].get(mode="promise_in_bounds")
        prev = jnp.where(iota == 0, prev_last, shifted)
        nr = keys[g] != prev
        if g == 0:
            nr = nr | (iota == 0)
        cand = jnp.where(nr, iota + g * _L, 0)
        seg = plsc.cummax(jnp.maximum(cand, carry_seg))
        carry_seg = jnp.max(seg)
        prev_last = jnp.max(keys[g])
        comp.append(seg * 256 + vals[g])

    comp, vals = _bitonic_sort(comp, vals)

    for g in range(_V):
        vals[g] = jnp.where(flag_v > 0, 200, vals[g])
    return vals


def _sc_body(ts_ref, table_ref, flag_ref, out_ref,
             ts_buf, table_buf, order_buf, stag0, stag1, flag_buf, sem_s):
    cid = lax.axis_index("c")
    sid = lax.axis_index("s")
    wid = sid * _NC + cid
    base = wid * _RPT

    pltpu.sync_copy(table_ref, table_buf)
    pltpu.sync_copy(ts_ref.at[pl.ds(base * _SEQ, _RPT * _SEQ)],
                    ts_buf.at[pl.ds(0, _RPT * _SEQ)])
    pltpu.sync_copy(flag_ref, flag_buf)

    iota = lax.iota(jnp.int32, _L)
    idxm1 = jnp.maximum(iota - 1, 0)
    flag_v = flag_buf[...]

    def issue_scatter(stag, row):
        pltpu.async_copy(stag.at[pl.ds(0, _SEQ)], out_ref.at[base + row],
                         sem_s)

    def wait_scatter():
        pltpu.make_async_copy(stag0.at[pl.ds(0, _SEQ)],
                              out_ref.at[base], sem_s).wait()

    def half(i, r, stag):
        vals = _row_order(ts_buf, r * _SEQ, iota, idxm1, flag_v)
        for g in range(13):
            order_buf[pl.ds(g * _L, _L)] = vals[g]

        @pl.when(i >= 1)
        def _():
            wait_scatter()

        def gather8(q, _):
            v = order_buf[pl.ds(q * 8, _L)]
            for u in range(8):
                p = q * 8 + u
                j = v[u]
                for m in range(4):
                    stag[p, pl.ds(m * _L, _L)] = table_buf[
                        pl.ds(j * _D + m * _L, _L)]
            return _

        lax.fori_loop(0, _SEQ // 8, gather8, None)
        issue_scatter(stag, r)

    def body(i, _):
        half(i, 2 * i, stag0)
        half(i, 2 * i + 1, stag1)
        return _

    lax.fori_loop(0, _RPT // 2, body, None)
    wait_scatter()
    wait_scatter()


@jax.jit
def kernel(ts, pos_emb):
    batch, seq = ts.shape
    assert seq == _SEQ and batch == _NW * _RPT
    table = jnp.pad(pos_emb, ((0, 256 - pos_emb.shape[0]), (0, 0)))
    flag = jnp.full((_L,), jnp.all(ts == 0.0).astype(jnp.int32))
    ts_flat = ts.reshape(-1)
    table_flat = table.reshape(-1)

    mesh = plsc.VectorSubcoreMesh(core_axis_name="c", subcore_axis_name="s")
    run = pl.kernel(
        _sc_body,
        out_type=jax.ShapeDtypeStruct((batch, _SEQ, _D), jnp.float32),
        mesh=mesh,
        compiler_params=pltpu.CompilerParams(
            needs_layout_passes=False, use_tc_tiling_on_sc=True),
        scratch_types=[
            pltpu.VMEM((_RPT * _SEQ + 8,), jnp.float32),
            pltpu.VMEM((256 * _D,), jnp.float32),
            pltpu.VMEM((256,), jnp.int32),
            pltpu.VMEM((208, _D), jnp.float32),
            pltpu.VMEM((208, _D), jnp.float32),
            pltpu.VMEM((_L,), jnp.int32),
            pltpu.SemaphoreType.DMA,
        ],
    )
    return run(ts_flat, table_flat, flag)


# speedup vs baseline: 2.2837x; 2.2837x over previous
import functools

import jax
import jax.numpy as jnp
from jax import lax
from jax.experimental import pallas as pl
from jax.experimental.pallas import tpu as pltpu
from jax.experimental.pallas import tpu_sc as plsc

_SEQ = 200
_D = 64
_L = 16
_V = 16
_NC = 2
_NS = 16
_NW = _NC * _NS
_RPT = 128


def _cmp_exchange(ka, va, kb, vb, asc):
    m = (ka <= kb) if asc else (ka >= kb)
    kl = jnp.where(m, ka, kb)
    vl = jnp.where(m, va, vb)
    kh = jnp.where(m, kb, ka)
    vh = jnp.where(m, vb, va)
    return kl, vl, kh, vh


def _bitonic_sort(keys, vals):
    for v in range(_V):
        keys[v], vals[v] = plsc.sort_key_val(keys[v], vals[v],
                                             descending=bool(v & 1))
    for vk in (2, 4, 8, 16):
        vj = vk // 2
        while vj >= 1:
            for v in range(_V):
                if (v & vj) == 0:
                    p = v | vj
                    asc = (v & vk) == 0
                    keys[v], vals[v], keys[p], vals[p] = _cmp_exchange(
                        keys[v], vals[v], keys[p], vals[p], asc)
            vj //= 2
        for v in range(_V):
            asc = (v & vk) == 0
            keys[v], vals[v] = plsc.sort_key_val(keys[v], vals[v],
                                                 descending=not asc)
    return keys, vals


def _row_order(ts_buf, row_base, iota, idxm1):
    inf = jnp.float32(jnp.inf)
    keys = []
    vals = []
    for g in range(_V):
        if g < 12:
            k = ts_buf[pl.ds(row_base + g * _L, _L)]
        elif g == 12:
            k = ts_buf[pl.ds(row_base + 12 * _L, _L)]
            k = jnp.where(iota < 8, k, inf)
        else:
            k = jnp.full((_L,), inf, jnp.float32)
        keys.append(k)
        vals.append(iota + g * _L)

    keys, vals = _bitonic_sort(keys, vals)

    comp = []
    carry_seg = jnp.int32(0)
    prev_last = jnp.float32(-jnp.inf)
    for g in range(_V):
        shifted = keys[g].at[idxm1].get(mode="promise_in_bounds")
        prev = jnp.where(iota == 0, prev_last, shifted)
        nr = keys[g] != prev
        if g == 0:
            nr = nr | (iota == 0)
        cand = jnp.where(nr, iota + g * _L, 0)
        seg = plsc.cummax(jnp.maximum(cand, carry_seg))
        carry_seg = jnp.max(seg)
        prev_last = jnp.max(keys[g])
        comp.append(seg * 256 + vals[g])

    comp, vals = _bitonic_sort(comp, vals)
    return vals


def _sc_body(ts_ref, table_ref, flag_ref, out_ref,
             ts_buf, table_buf, idx_lo, idx_hi, flag_buf, sem_s):
    wid = lax.axis_index("s") * _NC + lax.axis_index("c")
    base = wid * _RPT

    pltpu.sync_copy(ts_ref.at[pl.ds(base * _SEQ, _RPT * _SEQ)],
                    ts_buf.at[pl.ds(0, _RPT * _SEQ)])
    pltpu.sync_copy(table_ref, table_buf)
    pltpu.sync_copy(flag_ref, flag_buf)

    iota = lax.iota(jnp.int32, _L)
    idxm1 = jnp.maximum(iota - 1, 0)
    src_off = jnp.max(flag_buf[...]) * 256

    def body(r, _):
        b = r & 1
        vals = _row_order(ts_buf, r * _SEQ, iota, idxm1)

        @pl.when(r >= 2)
        def _wait():
            pltpu.make_async_copy(table_buf.at[pl.ds(0, 128)],
                                  out_ref.at[idx_lo.at[0]], sem_s).wait()
            pltpu.make_async_copy(table_buf.at[pl.ds(0, 72)],
                                  out_ref.at[idx_hi.at[0]], sem_s).wait()

        b_vec = jnp.full((_L,), 0, jnp.int32) + b
        rowbase = (base + r) * _SEQ
        for g in range(_V):
            value = iota + (g * _L) + rowbase
            j = vals[g]
            m_lo = j < 128
            m_hi = (j >= 128) & (j < _SEQ)
            plsc.store_scatter(idx_lo, [b_vec, j], value, mask=m_lo)
            plsc.store_scatter(idx_hi, [b_vec, j - 128], value, mask=m_hi)

        pltpu.async_copy(table_buf.at[pl.ds(src_off, 128)],
                         out_ref.at[idx_lo.at[b]], sem_s)
        pltpu.async_copy(table_buf.at[pl.ds(src_off + 128, 72)],
                         out_ref.at[idx_hi.at[b]], sem_s)
        return _

    lax.fori_loop(0, _RPT, body, None)
    for _ in range(2):
        pltpu.make_async_copy(table_buf.at[pl.ds(0, 128)],
                              out_ref.at[idx_lo.at[0]], sem_s).wait()
        pltpu.make_async_copy(table_buf.at[pl.ds(0, 72)],
                              out_ref.at[idx_hi.at[0]], sem_s).wait()


@jax.jit
def kernel(ts, pos_emb):
    batch, seq = ts.shape
    assert seq == _SEQ and batch == _NW * _RPT
    table = jnp.pad(pos_emb, ((0, 256 - pos_emb.shape[0]), (0, _D)))
    zero_blk = jnp.broadcast_to(jnp.pad(pos_emb[_SEQ], (0, _D)), (256, 2 * _D))
    table_aug = jnp.concatenate([table, zero_blk], axis=0)
    flag = jnp.full((_L,), jnp.all(ts == 0.0).astype(jnp.int32))
    ts_flat = ts.reshape(-1)

    mesh = plsc.VectorSubcoreMesh(core_axis_name="c", subcore_axis_name="s")
    run = pl.kernel(
        _sc_body,
        out_type=jax.ShapeDtypeStruct((batch * _SEQ, 2 * _D), jnp.float32),
        mesh=mesh,
        compiler_params=pltpu.CompilerParams(
            needs_layout_passes=False, use_tc_tiling_on_sc=False),
        scratch_types=[
            pltpu.VMEM((_RPT * _SEQ + 8,), jnp.float32),
            pltpu.VMEM((512, 2 * _D), jnp.float32),
            pltpu.VMEM((2, 128), jnp.int32),
            pltpu.VMEM((2, 72), jnp.int32),
            pltpu.VMEM((_L,), jnp.int32),
            pltpu.SemaphoreType.DMA,
        ],
    )
    out_wide = run(ts_flat, table_aug, flag)
    return out_wide.reshape(batch, _SEQ, 2 * _D)[..., :_D]
